# baseline (device time: 2903953 ns/iter reference)
import jax
import jax.numpy as jnp
from jax import lax
from jax.experimental import pallas as pl
from jax.experimental.pallas import tpu as pltpu

T_PER = 4096
D = 2048
F = 4096
E_LOCAL = 4

BT = 512
BF = 2048
CAP = 1536
CPB = CAP // BT

BW = 512
_NW1 = F // BW
_NCAST = E_LOCAL * _NW1

_VMEM_LIMIT = 60 * 1024 * 1024


def _peer_coords():
    return (1 - lax.axis_index("x"), lax.axis_index("y"), lax.axis_index("z"))


def _neighbor_barrier(peer):
    barrier_sem = pltpu.get_barrier_semaphore()
    pl.semaphore_signal(
        barrier_sem, inc=1, device_id=peer,
        device_id_type=pl.DeviceIdType.MESH,
    )
    pl.semaphore_wait(barrier_sem, 1)


def _exchange_and_cast(x_bf, a2d, W1, W2):

    def _rdmas(x_ref, a_ref, px_ref, pa_ref, sx, rx, sa, ra, peer):
        rdma_x = pltpu.make_async_remote_copy(
            src_ref=x_ref, dst_ref=px_ref, send_sem=sx, recv_sem=rx,
            device_id=peer, device_id_type=pl.DeviceIdType.MESH,
        )
        rdma_a = pltpu.make_async_remote_copy(
            src_ref=a_ref, dst_ref=pa_ref, send_sem=sa, recv_sem=ra,
            device_id=peer, device_id_type=pl.DeviceIdType.MESH,
        )
        return rdma_x, rdma_a

    def body(x_ref, a_ref, w1_ref, w2_ref,
             px_ref, pa_ref, w1o_ref, w2o_ref, sx, rx, sa, ra):
        g = pl.program_id(0)
        peer = _peer_coords()

        @pl.when(g == 0)
        def _():
            _neighbor_barrier(peer)
            rdma_x, rdma_a = _rdmas(
                x_ref, a_ref, px_ref, pa_ref, sx, rx, sa, ra, peer
            )
            rdma_x.start()
            rdma_a.start()

        w1o_ref[...] = w1_ref[...].astype(jnp.bfloat16)
        w2o_ref[...] = w2_ref[...].astype(jnp.bfloat16)

        @pl.when(g == _NCAST - 1)
        def _():
            rdma_x, rdma_a = _rdmas(
                x_ref, a_ref, px_ref, pa_ref, sx, rx, sa, ra, peer
            )
            rdma_x.wait()
            rdma_a.wait()

    return pl.pallas_call(
        body,
        grid=(_NCAST,),
        in_specs=[
            pl.BlockSpec((T_PER, D), lambda g: (0, 0)),
            pl.BlockSpec(a2d.shape, lambda g: (0, 0)),
            pl.BlockSpec((1, D, BW), lambda g: (g // _NW1, 0, g % _NW1)),
            pl.BlockSpec((1, BW, D), lambda g: (g // _NW1, g % _NW1, 0)),
        ],
        out_specs=(
            pl.BlockSpec((T_PER, D), lambda g: (0, 0)),
            pl.BlockSpec(a2d.shape, lambda g: (0, 0)),
            pl.BlockSpec((1, D, BW), lambda g: (g // _NW1, 0, g % _NW1)),
            pl.BlockSpec((1, BW, D), lambda g: (g // _NW1, g % _NW1, 0)),
        ),
        out_shape=(
            jax.ShapeDtypeStruct((T_PER, D), jnp.bfloat16),
            jax.ShapeDtypeStruct(a2d.shape, jnp.int32),
            jax.ShapeDtypeStruct((E_LOCAL, D, F), jnp.bfloat16),
            jax.ShapeDtypeStruct((E_LOCAL, F, D), jnp.bfloat16),
        ),
        scratch_shapes=[pltpu.SemaphoreType.DMA] * 4,
        compiler_params=pltpu.CompilerParams(
            collective_id=0, vmem_limit_bytes=_VMEM_LIMIT
        ),
    )(x_bf, a2d, W1, W2)


def _moe_routed(routed, w1, w2):
    n_tb = (E_LOCAL * CAP) // BT
    n_f = F // BF

    def body(t_ref, w1_ref, w2_ref, p_ref):
        f = pl.program_id(1)
        h = jnp.dot(t_ref[...], w1_ref[0], preferred_element_type=jnp.float32)
        h = jnp.maximum(h, 0.0).astype(jnp.bfloat16)
        p = jnp.dot(h, w2_ref[0], preferred_element_type=jnp.float32)

        @pl.when(f == 0)
        def _():
            p_ref[...] = p.astype(jnp.bfloat16)

        @pl.when(f > 0)
        def _():
            p_ref[...] += p.astype(jnp.bfloat16)

    return pl.pallas_call(
        body,
        grid=(n_tb, n_f),
        in_specs=[
            pl.BlockSpec((BT, D), lambda tb, f: (tb, 0)),
            pl.BlockSpec((1, D, BF), lambda tb, f: (tb // CPB, 0, f)),
            pl.BlockSpec((1, BF, D), lambda tb, f: (tb // CPB, f, 0)),
        ],
        out_specs=pl.BlockSpec((BT, D), lambda tb, f: (tb, 0)),
        out_shape=jax.ShapeDtypeStruct((E_LOCAL * CAP, D), jnp.bfloat16),
        compiler_params=pltpu.CompilerParams(vmem_limit_bytes=_VMEM_LIMIT),
    )(routed, w1, w2)


def _exchange_partials(peer_partial):

    def body(pp_ref, r_ref, send_sem, recv_sem):
        peer = _peer_coords()
        _neighbor_barrier(peer)

        rdma = pltpu.make_async_remote_copy(
            src_ref=pp_ref, dst_ref=r_ref,
            send_sem=send_sem, recv_sem=recv_sem,
            device_id=peer, device_id_type=pl.DeviceIdType.MESH,
        )
        rdma.start()
        rdma.wait()

    return pl.pallas_call(
        body,
        out_shape=jax.ShapeDtypeStruct((T_PER, D), jnp.bfloat16),
        in_specs=[pl.BlockSpec(memory_space=pltpu.VMEM)],
        out_specs=pl.BlockSpec(memory_space=pltpu.VMEM),
        scratch_shapes=[pltpu.SemaphoreType.DMA] * 2,
        compiler_params=pltpu.CompilerParams(
            collective_id=1, vmem_limit_bytes=_VMEM_LIMIT
        ),
    )(peer_partial)


def kernel(x, assign, W1, W2):
    my_x = lax.axis_index("x")

    x_bf = x.astype(jnp.bfloat16)
    a2d = assign.reshape(32, 128)

    peer_x, peer_a2d, w1, w2 = _exchange_and_cast(x_bf, a2d, W1, W2)

    all_assign = jnp.concatenate([assign, peer_a2d.reshape(-1)])
    lid = all_assign - E_LOCAL * my_x
    lid = jnp.where((lid >= 0) & (lid < E_LOCAL), lid, E_LOCAL)

    perm = jnp.argsort(lid)
    sorted_lid = lid[perm]
    starts = jnp.searchsorted(
        sorted_lid, jnp.arange(E_LOCAL + 1, dtype=jnp.int32)
    )
    counts = starts[1:] - starts[:-1]

    j = jnp.arange(E_LOCAL * CAP, dtype=jnp.int32)
    g = j // CAP
    r = j % CAP
    row_valid = r < counts[g]
    src = jnp.where(
        row_valid, perm[jnp.clip(starts[g] + r, 0, 2 * T_PER - 1)], 0
    )
    tokens = jnp.concatenate([x_bf, peer_x], axis=0)
    routed = tokens[src]

    p_routed = _moe_routed(routed, w1, w2)

    inv_perm = jnp.zeros_like(perm).at[perm].set(
        jnp.arange(2 * T_PER, dtype=jnp.int32)
    )
    rank = inv_perm - starts[lid]
    tok_valid = lid < E_LOCAL
    dest = jnp.where(tok_valid, lid * CAP + rank, 0)
    partial = jnp.where(tok_valid[:, None], p_routed[dest], 0)

    recv_partial = _exchange_partials(partial[T_PER:])
    return partial[:T_PER].astype(jnp.float32) + recv_partial.astype(
        jnp.float32
    )


# device time: 446777 ns/iter; 6.4998x vs baseline; 6.4998x over previous
import jax
import jax.numpy as jnp
from jax import lax
from jax.experimental import pallas as pl
from jax.experimental.pallas import tpu as pltpu

T_PER = 4096
D = 2048
F = 4096
E_LOCAL = 4

N_REP = 8
TS = T_PER // N_REP
BF = 1024

_VMEM_LIMIT = 60 * 1024 * 1024


def _xpeer_coords():
    return (1 - lax.axis_index("x"), lax.axis_index("y"), lax.axis_index("z"))


def _ring_pos():
    y = lax.axis_index("y")
    z = lax.axis_index("z")
    return jnp.where(y == 0, z, 7 - z)


def _ring_coords(p):
    p = jnp.remainder(p, N_REP)
    y = jnp.where(p < 4, 0, 1)
    z = jnp.where(p < 4, p, 7 - p)
    return (lax.axis_index("x"), y, z)


def _slice_of_ring_pos(p):
    p = jnp.remainder(p, N_REP)
    return jnp.where(p < 4, p, 11 - p)


def _exchange_slice(x_slice, a2d):

    def body(x_ref, a_ref, px_ref, pa_ref, sx, rx, sa, ra):
        peer = _xpeer_coords()
        barrier_sem = pltpu.get_barrier_semaphore()
        pl.semaphore_signal(
            barrier_sem, inc=1, device_id=peer,
            device_id_type=pl.DeviceIdType.MESH,
        )
        pl.semaphore_wait(barrier_sem, 1)

        rdma_x = pltpu.make_async_remote_copy(
            src_ref=x_ref, dst_ref=px_ref, send_sem=sx, recv_sem=rx,
            device_id=peer, device_id_type=pl.DeviceIdType.MESH,
        )
        rdma_a = pltpu.make_async_remote_copy(
            src_ref=a_ref, dst_ref=pa_ref, send_sem=sa, recv_sem=ra,
            device_id=peer, device_id_type=pl.DeviceIdType.MESH,
        )
        rdma_x.start()
        rdma_a.start()
        rdma_x.wait()
        rdma_a.wait()

    return pl.pallas_call(
        body,
        out_shape=(
            jax.ShapeDtypeStruct((TS, D), jnp.bfloat16),
            jax.ShapeDtypeStruct(a2d.shape, jnp.int32),
        ),
        in_specs=[
            pl.BlockSpec(memory_space=pltpu.VMEM),
            pl.BlockSpec(memory_space=pltpu.VMEM),
        ],
        out_specs=(
            pl.BlockSpec(memory_space=pltpu.VMEM),
            pl.BlockSpec(memory_space=pltpu.VMEM),
        ),
        scratch_shapes=[pltpu.SemaphoreType.DMA] * 4,
        compiler_params=pltpu.CompilerParams(
            collective_id=0, vmem_limit_bytes=_VMEM_LIMIT
        ),
    )(x_slice, a2d)


def _moe_dense(tokens, masks, W1, W2):
    n_f = F // BF

    def body(t_ref, m_ref, w1_ref, w2_ref, p_ref):
        e = pl.program_id(0)
        f = pl.program_id(1)

        w1 = w1_ref[0].astype(jnp.bfloat16)
        h = jnp.dot(t_ref[...], w1, preferred_element_type=jnp.float32)
        h = jnp.maximum(h, 0.0).astype(jnp.bfloat16)
        w2 = w2_ref[0].astype(jnp.bfloat16)
        p = jnp.dot(h, w2, preferred_element_type=jnp.float32)

        onehot = (
            lax.broadcasted_iota(jnp.int32, (1, E_LOCAL), 1) == e
        ).astype(jnp.float32)
        m = jnp.sum(
            m_ref[...].astype(jnp.float32) * onehot, axis=1, keepdims=True
        )
        contrib = (p * m).astype(jnp.bfloat16)

        @pl.when(jnp.logical_and(e == 0, f == 0))
        def _():
            p_ref[...] = contrib

        @pl.when(jnp.logical_or(e > 0, f > 0))
        def _():
            p_ref[...] += contrib

    return pl.pallas_call(
        body,
        grid=(E_LOCAL, n_f),
        in_specs=[
            pl.BlockSpec((2 * TS, D), lambda e, f: (0, 0)),
            pl.BlockSpec((2 * TS, E_LOCAL), lambda e, f: (0, 0)),
            pl.BlockSpec((1, D, BF), lambda e, f: (e, 0, f)),
            pl.BlockSpec((1, BF, D), lambda e, f: (e, f, 0)),
        ],
        out_specs=pl.BlockSpec((2 * TS, D), lambda e, f: (0, 0)),
        out_shape=jax.ShapeDtypeStruct((2 * TS, D), jnp.bfloat16),
        compiler_params=pltpu.CompilerParams(vmem_limit_bytes=_VMEM_LIMIT),
    )(tokens, masks, W1, W2)


def _combine_allgather(own_part, peer_part):

    def body(op_ref, pp_ref, out_ref, recvx, comm, sx, rx, ssems, rsems):
        rp = _ring_pos()
        xpeer = _xpeer_coords()
        right = _ring_coords(rp + 1)
        left = _ring_coords(rp - 1)

        barrier_sem = pltpu.get_barrier_semaphore()
        for nbr in (xpeer, left, right):
            pl.semaphore_signal(
                barrier_sem, inc=1, device_id=nbr,
                device_id_type=pl.DeviceIdType.MESH,
            )
        pl.semaphore_wait(barrier_sem, 3)

        rdma_x = pltpu.make_async_remote_copy(
            src_ref=pp_ref, dst_ref=recvx, send_sem=sx, recv_sem=rx,
            device_id=xpeer, device_id_type=pl.DeviceIdType.MESH,
        )
        rdma_x.start()
        rdma_x.wait()

        comm[0, :, :] = op_ref[...] + recvx[...]
        my_q = _slice_of_ring_pos(rp)
        out_ref[pl.ds(my_q * TS, TS), :] = comm[0, :, :].astype(jnp.float32)

        for h in range(N_REP - 1):
            rdma = pltpu.make_async_remote_copy(
                src_ref=comm.at[h],
                dst_ref=comm.at[h + 1],
                send_sem=ssems.at[h],
                recv_sem=rsems.at[h],
                device_id=right,
                device_id_type=pl.DeviceIdType.MESH,
            )
            rdma.start()
            rdma.wait()
            oq = _slice_of_ring_pos(rp - h - 1)
            out_ref[pl.ds(oq * TS, TS), :] = comm[h + 1, :, :].astype(
                jnp.float32
            )

    return pl.pallas_call(
        body,
        out_shape=jax.ShapeDtypeStruct((T_PER, D), jnp.float32),
        in_specs=[
            pl.BlockSpec(memory_space=pltpu.VMEM),
            pl.BlockSpec(memory_space=pltpu.VMEM),
        ],
        out_specs=pl.BlockSpec(memory_space=pltpu.VMEM),
        scratch_shapes=[
            pltpu.VMEM((TS, D), jnp.bfloat16),
            pltpu.VMEM((N_REP, TS, D), jnp.bfloat16),
            pltpu.SemaphoreType.DMA,
            pltpu.SemaphoreType.DMA,
            pltpu.SemaphoreType.DMA((N_REP - 1,)),
            pltpu.SemaphoreType.DMA((N_REP - 1,)),
        ],
        compiler_params=pltpu.CompilerParams(
            collective_id=1, vmem_limit_bytes=_VMEM_LIMIT
        ),
    )(own_part, peer_part)


def kernel(x, assign, W1, W2):
    my_x = lax.axis_index("x")
    q = 4 * lax.axis_index("y") + lax.axis_index("z")
    off = q * TS

    x_bf = x.astype(jnp.bfloat16)
    my_slice = lax.dynamic_slice(x_bf, (off, 0), (TS, D))
    my_a = lax.dynamic_slice(assign, (off,), (TS,))

    peer_slice, peer_a2d = _exchange_slice(my_slice, my_a.reshape(4, 128))

    tokens = jnp.concatenate([my_slice, peer_slice], axis=0)
    both_a = jnp.concatenate([my_a, peer_a2d.reshape(TS)])
    local_ids = jnp.arange(E_LOCAL, dtype=jnp.int32) + E_LOCAL * my_x
    masks = (both_a[:, None] == local_ids[None, :]).astype(jnp.bfloat16)

    partial = _moe_dense(tokens, masks, W1, W2)

    return _combine_allgather(partial[:TS], partial[TS:])


# device time: 370687 ns/iter; 7.8340x vs baseline; 1.2053x over previous
import jax
import jax.numpy as jnp
from jax import lax
from jax.experimental import pallas as pl
from jax.experimental.pallas import tpu as pltpu

T_PER = 4096
D = 2048
F = 4096
E_LOCAL = 4

N_REP = 8
TS = T_PER // N_REP
BF = 1024
N_R = 4
N_L = 3

_VMEM_LIMIT = 60 * 1024 * 1024


def _xpeer_coords():
    return (1 - lax.axis_index("x"), lax.axis_index("y"), lax.axis_index("z"))


def _ring_pos():
    y = lax.axis_index("y")
    z = lax.axis_index("z")
    return jnp.where(y == 0, z, 7 - z)


def _ring_coords(p):
    p = jnp.remainder(p, N_REP)
    y = jnp.where(p < 4, 0, 1)
    z = jnp.where(p < 4, p, 7 - p)
    return (lax.axis_index("x"), y, z)


def _slice_of_ring_pos(p):
    p = jnp.remainder(p, N_REP)
    return jnp.where(p < 4, p, 11 - p)


def _exchange_slice(x_slice, a2d):

    def body(x_ref, a_ref, px_ref, pa_ref, sx, rx, sa, ra):
        peer = _xpeer_coords()
        barrier_sem = pltpu.get_barrier_semaphore()
        pl.semaphore_signal(
            barrier_sem, inc=1, device_id=peer,
            device_id_type=pl.DeviceIdType.MESH,
        )
        pl.semaphore_wait(barrier_sem, 1)

        rdma_x = pltpu.make_async_remote_copy(
            src_ref=x_ref, dst_ref=px_ref, send_sem=sx, recv_sem=rx,
            device_id=peer, device_id_type=pl.DeviceIdType.MESH,
        )
        rdma_a = pltpu.make_async_remote_copy(
            src_ref=a_ref, dst_ref=pa_ref, send_sem=sa, recv_sem=ra,
            device_id=peer, device_id_type=pl.DeviceIdType.MESH,
        )
        rdma_x.start()
        rdma_a.start()
        rdma_x.wait()
        rdma_a.wait()

    return pl.pallas_call(
        body,
        out_shape=(
            jax.ShapeDtypeStruct((TS, D), jnp.bfloat16),
            jax.ShapeDtypeStruct(a2d.shape, jnp.int32),
        ),
        in_specs=[
            pl.BlockSpec(memory_space=pltpu.VMEM),
            pl.BlockSpec(memory_space=pltpu.VMEM),
        ],
        out_specs=(
            pl.BlockSpec(memory_space=pltpu.VMEM),
            pl.BlockSpec(memory_space=pltpu.VMEM),
        ),
        scratch_shapes=[pltpu.SemaphoreType.DMA] * 4,
        compiler_params=pltpu.CompilerParams(
            collective_id=0, vmem_limit_bytes=_VMEM_LIMIT
        ),
    )(x_slice, a2d)


def _moe_dense(own_tok, peer_tok, own_mask, peer_mask, W1, W2):
    n_f = F // BF

    def body(t1_ref, t2_ref, m1_ref, m2_ref, w1_ref, w2_ref,
             p1_ref, p2_ref):
        e = pl.program_id(0)
        f = pl.program_id(1)

        w1 = w1_ref[0].astype(jnp.bfloat16)
        w2 = w2_ref[0].astype(jnp.bfloat16)
        onehot = (
            lax.broadcasted_iota(jnp.int32, (1, E_LOCAL), 1) == e
        ).astype(jnp.float32)

        def ffn(t_ref, m_ref, p_ref):
            h = jnp.dot(t_ref[...], w1, preferred_element_type=jnp.float32)
            h = jnp.maximum(h, 0.0).astype(jnp.bfloat16)
            p = jnp.dot(h, w2, preferred_element_type=jnp.float32)
            m = jnp.sum(
                m_ref[...].astype(jnp.float32) * onehot, axis=1,
                keepdims=True,
            )
            contrib = (p * m).astype(jnp.bfloat16)

            @pl.when(jnp.logical_and(e == 0, f == 0))
            def _():
                p_ref[...] = contrib

            @pl.when(jnp.logical_or(e > 0, f > 0))
            def _():
                p_ref[...] += contrib

        ffn(t1_ref, m1_ref, p1_ref)
        ffn(t2_ref, m2_ref, p2_ref)

    tok_spec = pl.BlockSpec((TS, D), lambda e, f: (0, 0))
    mask_spec = pl.BlockSpec((TS, E_LOCAL), lambda e, f: (0, 0))
    return pl.pallas_call(
        body,
        grid=(E_LOCAL, n_f),
        in_specs=[
            tok_spec,
            tok_spec,
            mask_spec,
            mask_spec,
            pl.BlockSpec((1, D, BF), lambda e, f: (e, 0, f)),
            pl.BlockSpec((1, BF, D), lambda e, f: (e, f, 0)),
        ],
        out_specs=(tok_spec, tok_spec),
        out_shape=(
            jax.ShapeDtypeStruct((TS, D), jnp.bfloat16),
            jax.ShapeDtypeStruct((TS, D), jnp.bfloat16),
        ),
        compiler_params=pltpu.CompilerParams(vmem_limit_bytes=_VMEM_LIMIT),
    )(own_tok, peer_tok, own_mask, peer_mask, W1, W2)


def _combine_allgather(own_part, peer_part):

    def body(op_ref, pp_ref, out_ref, recvx, rcomm, lcomm,
             sx, rx, rssems, rrsems, lssems, lrsems):
        rp = _ring_pos()
        xpeer = _xpeer_coords()
        right = _ring_coords(rp + 1)
        left = _ring_coords(rp - 1)

        barrier_sem = pltpu.get_barrier_semaphore()
        for nbr in (xpeer, left, right):
            pl.semaphore_signal(
                barrier_sem, inc=1, device_id=nbr,
                device_id_type=pl.DeviceIdType.MESH,
            )
        pl.semaphore_wait(barrier_sem, 3)

        rdma_x = pltpu.make_async_remote_copy(
            src_ref=pp_ref, dst_ref=recvx, send_sem=sx, recv_sem=rx,
            device_id=xpeer, device_id_type=pl.DeviceIdType.MESH,
        )
        rdma_x.start()
        rdma_x.wait()

        s = op_ref[...] + recvx[...]
        rcomm[0, :, :] = s
        lcomm[0, :, :] = s
        my_q = _slice_of_ring_pos(rp)
        out_ref[pl.ds(my_q * TS, TS), :] = s.astype(jnp.float32)

        for h in range(N_R):
            rdma_r = pltpu.make_async_remote_copy(
                src_ref=rcomm.at[h],
                dst_ref=rcomm.at[h + 1],
                send_sem=rssems.at[h],
                recv_sem=rrsems.at[h],
                device_id=right,
                device_id_type=pl.DeviceIdType.MESH,
            )
            rdma_r.start()
            rdma_l = None
            if h < N_L:
                rdma_l = pltpu.make_async_remote_copy(
                    src_ref=lcomm.at[h],
                    dst_ref=lcomm.at[h + 1],
                    send_sem=lssems.at[h],
                    recv_sem=lrsems.at[h],
                    device_id=left,
                    device_id_type=pl.DeviceIdType.MESH,
                )
                rdma_l.start()

            rdma_r.wait()
            oq = _slice_of_ring_pos(rp - 1 - h)
            out_ref[pl.ds(oq * TS, TS), :] = rcomm[h + 1, :, :].astype(
                jnp.float32
            )
            if rdma_l is not None:
                rdma_l.wait()
                oq = _slice_of_ring_pos(rp + 1 + h)
                out_ref[pl.ds(oq * TS, TS), :] = lcomm[h + 1, :, :].astype(
                    jnp.float32
                )

    return pl.pallas_call(
        body,
        out_shape=jax.ShapeDtypeStruct((T_PER, D), jnp.float32),
        in_specs=[
            pl.BlockSpec(memory_space=pltpu.VMEM),
            pl.BlockSpec(memory_space=pltpu.VMEM),
        ],
        out_specs=pl.BlockSpec(memory_space=pltpu.VMEM),
        scratch_shapes=[
            pltpu.VMEM((TS, D), jnp.bfloat16),
            pltpu.VMEM((N_R + 1, TS, D), jnp.bfloat16),
            pltpu.VMEM((N_L + 1, TS, D), jnp.bfloat16),
            pltpu.SemaphoreType.DMA,
            pltpu.SemaphoreType.DMA,
            pltpu.SemaphoreType.DMA((N_R,)),
            pltpu.SemaphoreType.DMA((N_R,)),
            pltpu.SemaphoreType.DMA((N_L,)),
            pltpu.SemaphoreType.DMA((N_L,)),
        ],
        compiler_params=pltpu.CompilerParams(
            collective_id=1, vmem_limit_bytes=_VMEM_LIMIT
        ),
    )(own_part, peer_part)


def kernel(x, assign, W1, W2):
    my_x = lax.axis_index("x")
    q = 4 * lax.axis_index("y") + lax.axis_index("z")
    off = q * TS

    my_slice = lax.dynamic_slice(x, (off, 0), (TS, D)).astype(jnp.bfloat16)
    my_a = lax.dynamic_slice(assign, (off,), (TS,))

    peer_slice, peer_a2d = _exchange_slice(my_slice, my_a.reshape(4, 128))

    local_ids = jnp.arange(E_LOCAL, dtype=jnp.int32) + E_LOCAL * my_x
    own_mask = (my_a[:, None] == local_ids[None, :]).astype(jnp.bfloat16)
    peer_mask = (
        peer_a2d.reshape(TS)[:, None] == local_ids[None, :]
    ).astype(jnp.bfloat16)

    own_part, peer_part = _moe_dense(
        my_slice, peer_slice, own_mask, peer_mask, W1, W2
    )
    return _combine_allgather(own_part, peer_part)


# device time: 348392 ns/iter; 8.3353x vs baseline; 1.0640x over previous
import jax
import jax.numpy as jnp
from jax import lax
from jax.experimental import pallas as pl
from jax.experimental.pallas import tpu as pltpu

T_PER = 4096
D = 2048
F = 4096
E_LOCAL = 4

N_REP = 8
TS = T_PER // N_REP
BF = 1024
N_R = 4
N_L = 3
N_SUB = 2
SUB = TS // N_SUB

_VMEM_LIMIT = 60 * 1024 * 1024


def _xpeer_coords():
    return (1 - lax.axis_index("x"), lax.axis_index("y"), lax.axis_index("z"))


def _ring_pos():
    y = lax.axis_index("y")
    z = lax.axis_index("z")
    return jnp.where(y == 0, z, 7 - z)


def _ring_coords(p):
    p = jnp.remainder(p, N_REP)
    y = jnp.where(p < 4, 0, 1)
    z = jnp.where(p < 4, p, 7 - p)
    return (lax.axis_index("x"), y, z)


def _slice_of_ring_pos(p):
    p = jnp.remainder(p, N_REP)
    return jnp.where(p < 4, p, 11 - p)


def _exchange_slice(x_slice, a2d):

    def body(x_ref, a_ref, px_ref, pa_ref, sx, rx, sa, ra):
        peer = _xpeer_coords()
        barrier_sem = pltpu.get_barrier_semaphore()
        pl.semaphore_signal(
            barrier_sem, inc=1, device_id=peer,
            device_id_type=pl.DeviceIdType.MESH,
        )
        pl.semaphore_wait(barrier_sem, 1)

        rdma_x = pltpu.make_async_remote_copy(
            src_ref=x_ref, dst_ref=px_ref, send_sem=sx, recv_sem=rx,
            device_id=peer, device_id_type=pl.DeviceIdType.MESH,
        )
        rdma_a = pltpu.make_async_remote_copy(
            src_ref=a_ref, dst_ref=pa_ref, send_sem=sa, recv_sem=ra,
            device_id=peer, device_id_type=pl.DeviceIdType.MESH,
        )
        rdma_x.start()
        rdma_a.start()
        rdma_x.wait()
        rdma_a.wait()

    return pl.pallas_call(
        body,
        out_shape=(
            jax.ShapeDtypeStruct((TS, D), jnp.bfloat16),
            jax.ShapeDtypeStruct(a2d.shape, jnp.int32),
        ),
        in_specs=[
            pl.BlockSpec(memory_space=pltpu.VMEM),
            pl.BlockSpec(memory_space=pltpu.VMEM),
        ],
        out_specs=(
            pl.BlockSpec(memory_space=pltpu.VMEM),
            pl.BlockSpec(memory_space=pltpu.VMEM),
        ),
        scratch_shapes=[pltpu.SemaphoreType.DMA] * 4,
        compiler_params=pltpu.CompilerParams(
            collective_id=0, vmem_limit_bytes=_VMEM_LIMIT
        ),
    )(x_slice, a2d)


def _moe_dense(own_tok, peer_tok, own_mask, peer_mask, W1, W2):
    n_f = F // BF

    def body(t1_ref, t2_ref, m1_ref, m2_ref, w1_ref, w2_ref,
             p1_ref, p2_ref):
        e = pl.program_id(0)
        f = pl.program_id(1)

        w1 = w1_ref[0].astype(jnp.bfloat16)
        w2 = w2_ref[0].astype(jnp.bfloat16)
        onehot = (
            lax.broadcasted_iota(jnp.int32, (1, E_LOCAL), 1) == e
        ).astype(jnp.bfloat16)

        def ffn(t_ref, m_ref, p_ref):
            h = jnp.dot(t_ref[...], w1, preferred_element_type=jnp.float32)
            h = jnp.maximum(h, 0.0).astype(jnp.bfloat16)
            m = jnp.sum(m_ref[...] * onehot, axis=1, keepdims=True)
            h = h * m
            p = jnp.dot(h, w2, preferred_element_type=jnp.float32)
            contrib = p.astype(jnp.bfloat16)

            @pl.when(jnp.logical_and(e == 0, f == 0))
            def _():
                p_ref[...] = contrib

            @pl.when(jnp.logical_or(e > 0, f > 0))
            def _():
                p_ref[...] += contrib

        ffn(t1_ref, m1_ref, p1_ref)
        ffn(t2_ref, m2_ref, p2_ref)

    tok_spec = pl.BlockSpec((TS, D), lambda e, f: (0, 0))
    mask_spec = pl.BlockSpec((TS, E_LOCAL), lambda e, f: (0, 0))
    return pl.pallas_call(
        body,
        grid=(E_LOCAL, n_f),
        in_specs=[
            tok_spec,
            tok_spec,
            mask_spec,
            mask_spec,
            pl.BlockSpec((1, D, BF), lambda e, f: (e, 0, f)),
            pl.BlockSpec((1, BF, D), lambda e, f: (e, f, 0)),
        ],
        out_specs=(tok_spec, tok_spec),
        out_shape=(
            jax.ShapeDtypeStruct((TS, D), jnp.bfloat16),
            jax.ShapeDtypeStruct((TS, D), jnp.bfloat16),
        ),
        compiler_params=pltpu.CompilerParams(vmem_limit_bytes=_VMEM_LIMIT),
    )(own_tok, peer_tok, own_mask, peer_mask, W1, W2)


def _combine_allgather(own_part, peer_part):

    def _fwd(out_ref, src_q, dst_q, sub, ssem, rsem, target):
        rows_s = pl.ds(src_q * TS + sub * SUB, SUB)
        rows_d = pl.ds(dst_q * TS + sub * SUB, SUB)
        return pltpu.make_async_remote_copy(
            src_ref=out_ref.at[rows_s, :],
            dst_ref=out_ref.at[rows_d, :],
            send_sem=ssem,
            recv_sem=rsem,
            device_id=target,
            device_id_type=pl.DeviceIdType.MESH,
        )

    def body(op_ref, pp_ref, out_ref, recvx,
             sx, rx, rssems, rrsems, lssems, lrsems):
        rp = _ring_pos()
        xpeer = _xpeer_coords()
        right = _ring_coords(rp + 1)
        left = _ring_coords(rp - 1)

        barrier_sem = pltpu.get_barrier_semaphore()
        for nbr in (xpeer, left, right):
            pl.semaphore_signal(
                barrier_sem, inc=1, device_id=nbr,
                device_id_type=pl.DeviceIdType.MESH,
            )
        pl.semaphore_wait(barrier_sem, 3)

        rdma_x = pltpu.make_async_remote_copy(
            src_ref=pp_ref, dst_ref=recvx, send_sem=sx, recv_sem=rx,
            device_id=xpeer, device_id_type=pl.DeviceIdType.MESH,
        )
        rdma_x.start()
        rdma_x.wait()

        my_q = _slice_of_ring_pos(rp)
        out_ref[pl.ds(my_q * TS, TS), :] = op_ref[...] + recvx[...]

        sends = []
        r_descs = {}
        l_descs = {}
        for h in range(N_R):
            q_r = _slice_of_ring_pos(rp - h)
            for s in range(N_SUB):
                r_descs[(h, s)] = _fwd(
                    out_ref, q_r, q_r, s,
                    rssems.at[h, s], rrsems.at[h, s], right,
                )
        for h in range(N_L):
            q_l = _slice_of_ring_pos(rp + h)
            for s in range(N_SUB):
                l_descs[(h, s)] = _fwd(
                    out_ref, q_l, q_l, s,
                    lssems.at[h, s], lrsems.at[h, s], left,
                )

        for s in range(N_SUB):
            r_descs[(0, s)].start()
            l_descs[(0, s)].start()
            sends.append(r_descs[(0, s)])
            sends.append(l_descs[(0, s)])

        for h in range(1, N_R):
            for s in range(N_SUB):
                r_descs[(h - 1, s)].wait_recv()
                r_descs[(h, s)].start()
                sends.append(r_descs[(h, s)])
            if h < N_L:
                for s in range(N_SUB):
                    l_descs[(h - 1, s)].wait_recv()
                    l_descs[(h, s)].start()
                    sends.append(l_descs[(h, s)])

        for s in range(N_SUB):
            r_descs[(N_R - 1, s)].wait_recv()
            l_descs[(N_L - 1, s)].wait_recv()
        for d in sends:
            d.wait_send()

    return pl.pallas_call(
        body,
        out_shape=jax.ShapeDtypeStruct((T_PER, D), jnp.bfloat16),
        in_specs=[
            pl.BlockSpec(memory_space=pltpu.VMEM),
            pl.BlockSpec(memory_space=pltpu.VMEM),
        ],
        out_specs=pl.BlockSpec(memory_space=pltpu.VMEM),
        scratch_shapes=[
            pltpu.VMEM((TS, D), jnp.bfloat16),
            pltpu.SemaphoreType.DMA,
            pltpu.SemaphoreType.DMA,
            pltpu.SemaphoreType.DMA((N_R, N_SUB)),
            pltpu.SemaphoreType.DMA((N_R, N_SUB)),
            pltpu.SemaphoreType.DMA((N_L, N_SUB)),
            pltpu.SemaphoreType.DMA((N_L, N_SUB)),
        ],
        compiler_params=pltpu.CompilerParams(
            collective_id=1, vmem_limit_bytes=_VMEM_LIMIT
        ),
    )(own_part, peer_part)


def kernel(x, assign, W1, W2):
    my_x = lax.axis_index("x")
    q = 4 * lax.axis_index("y") + lax.axis_index("z")
    off = q * TS

    my_slice = lax.dynamic_slice(x, (off, 0), (TS, D)).astype(jnp.bfloat16)
    my_a = lax.dynamic_slice(assign, (off,), (TS,))

    peer_slice, peer_a2d = _exchange_slice(my_slice, my_a.reshape(4, 128))

    local_ids = jnp.arange(E_LOCAL, dtype=jnp.int32) + E_LOCAL * my_x
    own_mask = (my_a[:, None] == local_ids[None, :]).astype(jnp.bfloat16)
    peer_mask = (
        peer_a2d.reshape(TS)[:, None] == local_ids[None, :]
    ).astype(jnp.bfloat16)

    own_part, peer_part = _moe_dense(
        my_slice, peer_slice, own_mask, peer_mask, W1, W2
    )
    return _combine_allgather(own_part, peer_part)


# device time: 283974 ns/iter; 10.2261x vs baseline; 1.2268x over previous
import jax
import jax.numpy as jnp
from jax import lax
from jax.experimental import pallas as pl
from jax.experimental.pallas import tpu as pltpu

T_PER = 4096
D = 2048
F = 4096
E_LOCAL = 4

N_REP = 8
TS = T_PER // N_REP
BF = 1024
CAP_B = 256
NR_TOT = E_LOCAL * CAP_B
N_R = 4
N_L = 3
N_SUB = 2
SUB = TS // N_SUB

_VMEM_LIMIT = 64 * 1024 * 1024


def _xpeer_coords():
    return (1 - lax.axis_index("x"), lax.axis_index("y"), lax.axis_index("z"))


def _ring_pos():
    y = lax.axis_index("y")
    z = lax.axis_index("z")
    return jnp.where(y == 0, z, 7 - z)


def _ring_coords(p):
    p = jnp.remainder(p, N_REP)
    y = jnp.where(p < 4, 0, 1)
    z = jnp.where(p < 4, p, 7 - p)
    return (lax.axis_index("x"), y, z)


def _slice_of_ring_pos(p):
    p = jnp.remainder(p, N_REP)
    return jnp.where(p < 4, p, 11 - p)


def _exchange_slice(x_slice, a2d):

    def body(x_ref, a_ref, px_ref, pa_ref, sx, rx, sa, ra):
        peer = _xpeer_coords()
        barrier_sem = pltpu.get_barrier_semaphore()
        pl.semaphore_signal(
            barrier_sem, inc=1, device_id=peer,
            device_id_type=pl.DeviceIdType.MESH,
        )
        pl.semaphore_wait(barrier_sem, 1)

        rdma_x = pltpu.make_async_remote_copy(
            src_ref=x_ref, dst_ref=px_ref, send_sem=sx, recv_sem=rx,
            device_id=peer, device_id_type=pl.DeviceIdType.MESH,
        )
        rdma_a = pltpu.make_async_remote_copy(
            src_ref=a_ref, dst_ref=pa_ref, send_sem=sa, recv_sem=ra,
            device_id=peer, device_id_type=pl.DeviceIdType.MESH,
        )
        rdma_x.start()
        rdma_a.start()
        rdma_x.wait()
        rdma_a.wait()

    return pl.pallas_call(
        body,
        out_shape=(
            jax.ShapeDtypeStruct((TS, D), jnp.bfloat16),
            jax.ShapeDtypeStruct(a2d.shape, jnp.int32),
        ),
        in_specs=[
            pl.BlockSpec(memory_space=pltpu.VMEM),
            pl.BlockSpec(memory_space=pltpu.VMEM),
        ],
        out_specs=(
            pl.BlockSpec(memory_space=pltpu.VMEM),
            pl.BlockSpec(memory_space=pltpu.VMEM),
        ),
        scratch_shapes=[pltpu.SemaphoreType.DMA] * 4,
        compiler_params=pltpu.CompilerParams(
            collective_id=0, vmem_limit_bytes=_VMEM_LIMIT
        ),
    )(x_slice, a2d)


def _moe_routed(own_tok, peer_tok, P1, P2, U1, U2, W1, W2):
    n_f = F // BF

    def body(t1_ref, t2_ref, p1_in, p2_in, u1_ref, u2_ref,
             w1_ref, w2_ref, p1_ref, p2_ref, routed, acc):
        e = pl.program_id(0)
        f = pl.program_id(1)

        @pl.when(jnp.logical_and(e == 0, f == 0))
        def _():
            r = jnp.dot(
                p1_in[...], t1_ref[...], preferred_element_type=jnp.float32
            ) + jnp.dot(
                p2_in[...], t2_ref[...], preferred_element_type=jnp.float32
            )
            routed[...] = r.astype(jnp.bfloat16)

        w1 = w1_ref[0].astype(jnp.bfloat16)
        w2 = w2_ref[0].astype(jnp.bfloat16)
        slab = routed[pl.ds(e * CAP_B, CAP_B), :]
        h = jnp.dot(slab, w1, preferred_element_type=jnp.float32)
        h = jnp.maximum(h, 0.0).astype(jnp.bfloat16)
        p = jnp.dot(h, w2, preferred_element_type=jnp.float32)

        @pl.when(f == 0)
        def _():
            acc[pl.ds(e * CAP_B, CAP_B), :] = p.astype(jnp.bfloat16)

        @pl.when(f > 0)
        def _():
            acc[pl.ds(e * CAP_B, CAP_B), :] += p.astype(jnp.bfloat16)

        @pl.when(jnp.logical_and(e == E_LOCAL - 1, f == n_f - 1))
        def _():
            p1_ref[...] = jnp.dot(
                u1_ref[...], acc[...], preferred_element_type=jnp.float32
            ).astype(jnp.bfloat16)
            p2_ref[...] = jnp.dot(
                u2_ref[...], acc[...], preferred_element_type=jnp.float32
            ).astype(jnp.bfloat16)

    tok_spec = pl.BlockSpec((TS, D), lambda e, f: (0, 0))
    route_spec = pl.BlockSpec((NR_TOT, TS), lambda e, f: (0, 0))
    unroute_spec = pl.BlockSpec((TS, NR_TOT), lambda e, f: (0, 0))
    return pl.pallas_call(
        body,
        grid=(E_LOCAL, n_f),
        in_specs=[
            tok_spec,
            tok_spec,
            route_spec,
            route_spec,
            unroute_spec,
            unroute_spec,
            pl.BlockSpec((1, D, BF), lambda e, f: (e, 0, f)),
            pl.BlockSpec((1, BF, D), lambda e, f: (e, f, 0)),
        ],
        out_specs=(tok_spec, tok_spec),
        out_shape=(
            jax.ShapeDtypeStruct((TS, D), jnp.bfloat16),
            jax.ShapeDtypeStruct((TS, D), jnp.bfloat16),
        ),
        scratch_shapes=[
            pltpu.VMEM((NR_TOT, D), jnp.bfloat16),
            pltpu.VMEM((NR_TOT, D), jnp.bfloat16),
        ],
        compiler_params=pltpu.CompilerParams(vmem_limit_bytes=_VMEM_LIMIT),
    )(own_tok, peer_tok, P1, P2, U1, U2, W1, W2)


def _combine_allgather(own_part, peer_part):

    def _fwd(out_ref, src_q, dst_q, sub, ssem, rsem, target):
        rows_s = pl.ds(src_q * TS + sub * SUB, SUB)
        rows_d = pl.ds(dst_q * TS + sub * SUB, SUB)
        return pltpu.make_async_remote_copy(
            src_ref=out_ref.at[rows_s, :],
            dst_ref=out_ref.at[rows_d, :],
            send_sem=ssem,
            recv_sem=rsem,
            device_id=target,
            device_id_type=pl.DeviceIdType.MESH,
        )

    def body(op_ref, pp_ref, out_ref, recvx,
             sx, rx, rssems, rrsems, lssems, lrsems):
        rp = _ring_pos()
        xpeer = _xpeer_coords()
        right = _ring_coords(rp + 1)
        left = _ring_coords(rp - 1)

        barrier_sem = pltpu.get_barrier_semaphore()
        for nbr in (xpeer, left, right):
            pl.semaphore_signal(
                barrier_sem, inc=1, device_id=nbr,
                device_id_type=pl.DeviceIdType.MESH,
            )
        pl.semaphore_wait(barrier_sem, 3)

        rdma_x = pltpu.make_async_remote_copy(
            src_ref=pp_ref, dst_ref=recvx, send_sem=sx, recv_sem=rx,
            device_id=xpeer, device_id_type=pl.DeviceIdType.MESH,
        )
        rdma_x.start()
        rdma_x.wait()

        my_q = _slice_of_ring_pos(rp)
        out_ref[pl.ds(my_q * TS, TS), :] = op_ref[...] + recvx[...]

        sends = []
        r_descs = {}
        l_descs = {}
        for h in range(N_R):
            q_r = _slice_of_ring_pos(rp - h)
            for s in range(N_SUB):
                r_descs[(h, s)] = _fwd(
                    out_ref, q_r, q_r, s,
                    rssems.at[h, s], rrsems.at[h, s], right,
                )
        for h in range(N_L):
            q_l = _slice_of_ring_pos(rp + h)
            for s in range(N_SUB):
                l_descs[(h, s)] = _fwd(
                    out_ref, q_l, q_l, s,
                    lssems.at[h, s], lrsems.at[h, s], left,
                )

        for s in range(N_SUB):
            r_descs[(0, s)].start()
            l_descs[(0, s)].start()
            sends.append(r_descs[(0, s)])
            sends.append(l_descs[(0, s)])

        for h in range(1, N_R):
            for s in range(N_SUB):
                r_descs[(h - 1, s)].wait_recv()
                r_descs[(h, s)].start()
                sends.append(r_descs[(h, s)])
            if h < N_L:
                for s in range(N_SUB):
                    l_descs[(h - 1, s)].wait_recv()
                    l_descs[(h, s)].start()
                    sends.append(l_descs[(h, s)])

        for s in range(N_SUB):
            r_descs[(N_R - 1, s)].wait_recv()
            l_descs[(N_L - 1, s)].wait_recv()
        for d in sends:
            d.wait_send()

    return pl.pallas_call(
        body,
        out_shape=jax.ShapeDtypeStruct((T_PER, D), jnp.bfloat16),
        in_specs=[
            pl.BlockSpec(memory_space=pltpu.VMEM),
            pl.BlockSpec(memory_space=pltpu.VMEM),
        ],
        out_specs=pl.BlockSpec(memory_space=pltpu.VMEM),
        scratch_shapes=[
            pltpu.VMEM((TS, D), jnp.bfloat16),
            pltpu.SemaphoreType.DMA,
            pltpu.SemaphoreType.DMA,
            pltpu.SemaphoreType.DMA((N_R, N_SUB)),
            pltpu.SemaphoreType.DMA((N_R, N_SUB)),
            pltpu.SemaphoreType.DMA((N_L, N_SUB)),
            pltpu.SemaphoreType.DMA((N_L, N_SUB)),
        ],
        compiler_params=pltpu.CompilerParams(
            collective_id=1, vmem_limit_bytes=_VMEM_LIMIT
        ),
    )(own_part, peer_part)


def kernel(x, assign, W1, W2):
    my_x = lax.axis_index("x")
    q = 4 * lax.axis_index("y") + lax.axis_index("z")
    off = q * TS

    my_slice = lax.dynamic_slice(x, (off, 0), (TS, D)).astype(jnp.bfloat16)
    my_a = lax.dynamic_slice(assign, (off,), (TS,))

    peer_slice, peer_a2d = _exchange_slice(my_slice, my_a.reshape(4, 128))

    both_a = jnp.concatenate([my_a, peer_a2d.reshape(TS)])
    lid = both_a - E_LOCAL * my_x
    valid = (lid >= 0) & (lid < E_LOCAL)
    lid5 = jnp.where(valid, lid, E_LOCAL)
    onehot = (
        lid5[:, None] == jnp.arange(E_LOCAL + 1, dtype=jnp.int32)
    ).astype(jnp.int32)
    rank = jnp.sum(onehot * jnp.cumsum(onehot, axis=0), axis=1) - 1
    dest = jnp.where(
        valid & (rank < CAP_B), lid5 * CAP_B + rank, NR_TOT
    )

    rows = jnp.arange(NR_TOT, dtype=jnp.int32)
    P = (dest[None, :] == rows[:, None]).astype(jnp.bfloat16)
    U = (dest[:, None] == rows[None, :]).astype(jnp.bfloat16)

    own_part, peer_part = _moe_routed(
        my_slice, peer_slice,
        P[:, :TS], P[:, TS:], U[:TS], U[TS:], W1, W2,
    )
    return _combine_allgather(own_part, peer_part)


# device time: 267640 ns/iter; 10.8502x vs baseline; 1.0610x over previous
import jax
import jax.numpy as jnp
from jax import lax
from jax.experimental import pallas as pl
from jax.experimental.pallas import tpu as pltpu

T_PER = 4096
D = 2048
F = 4096
E_LOCAL = 4

N_REP = 8
TS = T_PER // N_REP
BF = 1024
CAP_B = 256
NR_TOT = E_LOCAL * CAP_B
N_R = 4
N_L = 3
N_SUB = 4
SUB = TS // N_SUB

_VMEM_LIMIT = 64 * 1024 * 1024


def _xpeer_coords():
    return (1 - lax.axis_index("x"), lax.axis_index("y"), lax.axis_index("z"))


def _ring_pos():
    y = lax.axis_index("y")
    z = lax.axis_index("z")
    return jnp.where(y == 0, z, 7 - z)


def _ring_coords(p):
    p = jnp.remainder(p, N_REP)
    y = jnp.where(p < 4, 0, 1)
    z = jnp.where(p < 4, p, 7 - p)
    return (lax.axis_index("x"), y, z)


def _slice_of_ring_pos(p):
    p = jnp.remainder(p, N_REP)
    return jnp.where(p < 4, p, 11 - p)


def _exchange_slice(x_slice, a2d):

    def body(x_ref, a_ref, px_ref, pa_ref, sx, rx, sa, ra):
        peer = _xpeer_coords()
        barrier_sem = pltpu.get_barrier_semaphore()
        pl.semaphore_signal(
            barrier_sem, inc=1, device_id=peer,
            device_id_type=pl.DeviceIdType.MESH,
        )
        pl.semaphore_wait(barrier_sem, 1)

        rdma_x = pltpu.make_async_remote_copy(
            src_ref=x_ref, dst_ref=px_ref, send_sem=sx, recv_sem=rx,
            device_id=peer, device_id_type=pl.DeviceIdType.MESH,
        )
        rdma_a = pltpu.make_async_remote_copy(
            src_ref=a_ref, dst_ref=pa_ref, send_sem=sa, recv_sem=ra,
            device_id=peer, device_id_type=pl.DeviceIdType.MESH,
        )
        rdma_x.start()
        rdma_a.start()
        rdma_x.wait()
        rdma_a.wait()

    return pl.pallas_call(
        body,
        out_shape=(
            jax.ShapeDtypeStruct((TS, D), jnp.bfloat16),
            jax.ShapeDtypeStruct(a2d.shape, jnp.int32),
        ),
        in_specs=[
            pl.BlockSpec(memory_space=pltpu.VMEM),
            pl.BlockSpec(memory_space=pltpu.VMEM),
        ],
        out_specs=(
            pl.BlockSpec(memory_space=pltpu.VMEM),
            pl.BlockSpec(memory_space=pltpu.VMEM),
        ),
        scratch_shapes=[pltpu.SemaphoreType.DMA] * 4,
        compiler_params=pltpu.CompilerParams(
            collective_id=0, vmem_limit_bytes=_VMEM_LIMIT
        ),
    )(x_slice, a2d)


def _moe_routed(own_tok, peer_tok, P1, P2, U1, U2, W1, W2):
    n_f = F // BF

    def body(t1_ref, t2_ref, p1_in, p2_in, u1_ref, u2_ref,
             w1_ref, w2_ref, p1_ref, p2_ref, routed, acc):
        e = pl.program_id(0)
        f = pl.program_id(1)

        @pl.when(jnp.logical_and(e == 0, f == 0))
        def _():
            r = jnp.dot(
                p1_in[...], t1_ref[...], preferred_element_type=jnp.float32
            ) + jnp.dot(
                p2_in[...], t2_ref[...], preferred_element_type=jnp.float32
            )
            routed[...] = r.astype(jnp.bfloat16)

        w1 = w1_ref[0].astype(jnp.bfloat16)
        w2 = w2_ref[0].astype(jnp.bfloat16)
        slab = routed[pl.ds(e * CAP_B, CAP_B), :]
        h = jnp.dot(slab, w1, preferred_element_type=jnp.float32)
        h = jnp.maximum(h, 0.0).astype(jnp.bfloat16)
        p = jnp.dot(h, w2, preferred_element_type=jnp.float32)

        @pl.when(f == 0)
        def _():
            acc[pl.ds(e * CAP_B, CAP_B), :] = p.astype(jnp.bfloat16)

        @pl.when(f > 0)
        def _():
            acc[pl.ds(e * CAP_B, CAP_B), :] += p.astype(jnp.bfloat16)

        @pl.when(jnp.logical_and(e == E_LOCAL - 1, f == n_f - 1))
        def _():
            p1_ref[...] = jnp.dot(
                u1_ref[...], acc[...], preferred_element_type=jnp.float32
            ).astype(jnp.bfloat16)
            p2_ref[...] = jnp.dot(
                u2_ref[...], acc[...], preferred_element_type=jnp.float32
            ).astype(jnp.bfloat16)

    tok_spec = pl.BlockSpec((TS, D), lambda e, f: (0, 0))
    route_spec = pl.BlockSpec((NR_TOT, TS), lambda e, f: (0, 0))
    unroute_spec = pl.BlockSpec((TS, NR_TOT), lambda e, f: (0, 0))
    return pl.pallas_call(
        body,
        grid=(E_LOCAL, n_f),
        in_specs=[
            tok_spec,
            tok_spec,
            route_spec,
            route_spec,
            unroute_spec,
            unroute_spec,
            pl.BlockSpec((1, D, BF), lambda e, f: (e, 0, f)),
            pl.BlockSpec((1, BF, D), lambda e, f: (e, f, 0)),
        ],
        out_specs=(tok_spec, tok_spec),
        out_shape=(
            jax.ShapeDtypeStruct((TS, D), jnp.bfloat16),
            jax.ShapeDtypeStruct((TS, D), jnp.bfloat16),
        ),
        scratch_shapes=[
            pltpu.VMEM((NR_TOT, D), jnp.bfloat16),
            pltpu.VMEM((NR_TOT, D), jnp.bfloat16),
        ],
        compiler_params=pltpu.CompilerParams(vmem_limit_bytes=_VMEM_LIMIT),
    )(own_tok, peer_tok, P1, P2, U1, U2, W1, W2)


def _combine_allgather(own_part, peer_part):

    def _fwd(out_ref, src_q, dst_q, sub, ssem, rsem, target):
        rows_s = pl.ds(src_q * TS + sub * SUB, SUB)
        rows_d = pl.ds(dst_q * TS + sub * SUB, SUB)
        return pltpu.make_async_remote_copy(
            src_ref=out_ref.at[rows_s, :],
            dst_ref=out_ref.at[rows_d, :],
            send_sem=ssem,
            recv_sem=rsem,
            device_id=target,
            device_id_type=pl.DeviceIdType.MESH,
        )

    def body(op_ref, pp_ref, out_ref, recvx,
             sxs, rxs, rssems, rrsems, lssems, lrsems):
        rp = _ring_pos()
        xpeer = _xpeer_coords()
        right = _ring_coords(rp + 1)
        left = _ring_coords(rp - 1)

        barrier_sem = pltpu.get_barrier_semaphore()
        for nbr in (xpeer, left, right):
            pl.semaphore_signal(
                barrier_sem, inc=1, device_id=nbr,
                device_id_type=pl.DeviceIdType.MESH,
            )
        pl.semaphore_wait(barrier_sem, 3)

        x_descs = []
        for s in range(N_SUB):
            rows = pl.ds(s * SUB, SUB)
            d = pltpu.make_async_remote_copy(
                src_ref=pp_ref.at[rows, :], dst_ref=recvx.at[rows, :],
                send_sem=sxs.at[s], recv_sem=rxs.at[s],
                device_id=xpeer, device_id_type=pl.DeviceIdType.MESH,
            )
            d.start()
            x_descs.append(d)

        sends = []
        r_descs = {}
        l_descs = {}
        for h in range(N_R):
            q_r = _slice_of_ring_pos(rp - h)
            for s in range(N_SUB):
                r_descs[(h, s)] = _fwd(
                    out_ref, q_r, q_r, s,
                    rssems.at[h, s], rrsems.at[h, s], right,
                )
        for h in range(N_L):
            q_l = _slice_of_ring_pos(rp + h)
            for s in range(N_SUB):
                l_descs[(h, s)] = _fwd(
                    out_ref, q_l, q_l, s,
                    lssems.at[h, s], lrsems.at[h, s], left,
                )

        my_q = _slice_of_ring_pos(rp)
        for s in range(N_SUB):
            rows = pl.ds(s * SUB, SUB)
            x_descs[s].wait_recv()
            out_ref[pl.ds(my_q * TS + s * SUB, SUB), :] = (
                op_ref[rows, :] + recvx[rows, :]
            )
            r_descs[(0, s)].start()
            l_descs[(0, s)].start()
            sends.append(r_descs[(0, s)])
            sends.append(l_descs[(0, s)])

        for h in range(1, N_R):
            for s in range(N_SUB):
                r_descs[(h - 1, s)].wait_recv()
                r_descs[(h, s)].start()
                sends.append(r_descs[(h, s)])
            if h < N_L:
                for s in range(N_SUB):
                    l_descs[(h - 1, s)].wait_recv()
                    l_descs[(h, s)].start()
                    sends.append(l_descs[(h, s)])

        for s in range(N_SUB):
            r_descs[(N_R - 1, s)].wait_recv()
            l_descs[(N_L - 1, s)].wait_recv()
        for d in x_descs:
            d.wait_send()
        for d in sends:
            d.wait_send()

    return pl.pallas_call(
        body,
        out_shape=jax.ShapeDtypeStruct((T_PER, D), jnp.bfloat16),
        in_specs=[
            pl.BlockSpec(memory_space=pltpu.VMEM),
            pl.BlockSpec(memory_space=pltpu.VMEM),
        ],
        out_specs=pl.BlockSpec(memory_space=pltpu.VMEM),
        scratch_shapes=[
            pltpu.VMEM((TS, D), jnp.bfloat16),
            pltpu.SemaphoreType.DMA((N_SUB,)),
            pltpu.SemaphoreType.DMA((N_SUB,)),
            pltpu.SemaphoreType.DMA((N_R, N_SUB)),
            pltpu.SemaphoreType.DMA((N_R, N_SUB)),
            pltpu.SemaphoreType.DMA((N_L, N_SUB)),
            pltpu.SemaphoreType.DMA((N_L, N_SUB)),
        ],
        compiler_params=pltpu.CompilerParams(
            collective_id=1, vmem_limit_bytes=_VMEM_LIMIT
        ),
    )(own_part, peer_part)


def kernel(x, assign, W1, W2):
    my_x = lax.axis_index("x")
    q = 4 * lax.axis_index("y") + lax.axis_index("z")
    off = q * TS

    my_slice = lax.dynamic_slice(x, (off, 0), (TS, D)).astype(jnp.bfloat16)
    my_a = lax.dynamic_slice(assign, (off,), (TS,))

    peer_slice, peer_a2d = _exchange_slice(my_slice, my_a.reshape(4, 128))

    both_a = jnp.concatenate([my_a, peer_a2d.reshape(TS)])
    lid = both_a - E_LOCAL * my_x
    valid = (lid >= 0) & (lid < E_LOCAL)
    lid5 = jnp.where(valid, lid, E_LOCAL)
    onehot = (
        lid5[:, None] == jnp.arange(E_LOCAL + 1, dtype=jnp.int32)
    ).astype(jnp.int32)
    rank = jnp.sum(onehot * jnp.cumsum(onehot, axis=0), axis=1) - 1
    dest = jnp.where(
        valid & (rank < CAP_B), lid5 * CAP_B + rank, NR_TOT
    )

    rows = jnp.arange(NR_TOT, dtype=jnp.int32)
    P = (dest[None, :] == rows[:, None]).astype(jnp.bfloat16)
    U = (dest[:, None] == rows[None, :]).astype(jnp.bfloat16)

    own_part, peer_part = _moe_routed(
        my_slice, peer_slice,
        P[:, :TS], P[:, TS:], U[:TS], U[TS:], W1, W2,
    )
    return _combine_allgather(own_part, peer_part)
